# baseline (device time: 272141 ns/iter reference)
import jax
import jax.numpy as jnp
from jax import lax
from jax.experimental import pallas as pl
from jax.experimental.pallas import tpu as pltpu

N_DEV = 8
HQ_LOC = 8
DH = 128
SQ = 1024
SKV = 1024
D_MODEL = 1024
SCALE = 0.08838834764831843


def kernel(x, Wq, K_ext, V_ext, Wo):
    pos = lax.axis_index("i")
    x2 = x[0]
    k_loc = lax.dynamic_slice_in_dim(K_ext[0], pos * HQ_LOC, HQ_LOC, axis=1)
    v_loc = lax.dynamic_slice_in_dim(V_ext[0], pos * HQ_LOC, HQ_LOC, axis=1)

    def body(x_ref, wq_ref, k_ref, v_ref, wo_ref, out_ref,
             ctx_ref, stage_ref, p1_send, p1_recv, p2_send, p2_recv):
        my = lax.axis_index("i")

        barrier_sem = pltpu.get_barrier_semaphore()
        for t in range(1, N_DEV):
            pl.semaphore_signal(
                barrier_sem, inc=1,
                device_id=(lax.rem(my + t, N_DEV),),
                device_id_type=pl.DeviceIdType.MESH,
            )
        pl.semaphore_wait(barrier_sem, N_DEV - 1)

        bf = jnp.bfloat16
        q = jnp.dot(x_ref[...].astype(bf), wq_ref[...].astype(bf),
                    preferred_element_type=jnp.float32) * SCALE

        CHB = SQ // N_DEV
        NEG = jnp.float32(-1e9)
        qi_l = lax.broadcasted_iota(jnp.int32, (CHB, CHB), 0)
        ki_l = lax.broadcasted_iota(jnp.int32, (CHB, CHB), 1)
        t_low = jnp.where(qi_l <= ki_l, 0.0, NEG)
        t_high = jnp.where(ki_l <= qi_l, 0.0, NEG)
        t_glob = jnp.where(ki_l < 32, 0.0, NEG)
        t_1_0 = jnp.where((qi_l <= ki_l) | (ki_l < 32), 0.0, NEG)
        qi_0 = lax.broadcasted_iota(jnp.int32, (CHB, SKV), 0)
        ki_0 = lax.broadcasted_iota(jnp.int32, (CHB, SKV), 1)
        b_0 = jnp.where(
            (jnp.abs(qi_0 - ki_0) <= 128) | (ki_0 < 32) | (qi_0 < 32),
            0.0, NEG)

        def key_blocks(qb):
            if qb == 0:
                return None
            kbs = [(0, t_1_0 if qb == 1 else t_glob)]
            if qb >= 2:
                kbs.append((qb - 1, t_low))
            kbs.append((qb, None))
            if qb + 1 < N_DEV:
                kbs.append((qb + 1, t_high))
            return kbs

        sends = []
        for j in range(N_DEV):
            r0 = j * CHB
            for h in range(HQ_LOC):
                q_b = q[r0:r0 + CHB, h * DH:(h + 1) * DH].astype(bf)
                k_h = k_ref[:, h, :]
                v_h = v_ref[:, h, :]
                if j == 0:
                    s = lax.dot_general(
                        q_b, k_h.astype(bf), (((1,), (1,)), ((), ())),
                        preferred_element_type=jnp.float32) + b_0
                    m = jnp.max(s, axis=-1, keepdims=True)
                    w = jnp.exp(s - m)
                    den = jnp.sum(w, axis=-1, keepdims=True)
                    acc = jnp.dot(w.astype(bf), v_h.astype(bf),
                                  preferred_element_type=jnp.float32)
                else:
                    ss = []
                    for kb, bias in key_blocks(j):
                        s_t = lax.dot_general(
                            q_b, k_h[kb * CHB:(kb + 1) * CHB, :].astype(bf),
                            (((1,), (1,)), ((), ())),
                            preferred_element_type=jnp.float32)
                        ss.append(s_t if bias is None else s_t + bias)
                    m = ss[0].max(axis=-1, keepdims=True)
                    for s_t in ss[1:]:
                        m = jnp.maximum(m, s_t.max(axis=-1, keepdims=True))
                    den = jnp.float32(0.0)
                    acc = jnp.float32(0.0)
                    for (kb, _), s_t in zip(key_blocks(j), ss):
                        e = jnp.exp(s_t - m)
                        den = den + jnp.sum(e, axis=-1, keepdims=True)
                        acc = acc + jnp.dot(
                            e.astype(bf), v_h[kb * CHB:(kb + 1) * CHB, :].astype(bf),
                            preferred_element_type=jnp.float32)
                ctx_ref[r0:r0 + CHB, h * DH:(h + 1) * DH] = acc * (1.0 / den)

            out_ref[r0:r0 + CHB, :] = jnp.dot(
                ctx_ref[r0:r0 + CHB, :].astype(bf), wo_ref[...].astype(bf),
                preferred_element_type=jnp.float32)
            t_dyn = lax.rem(j - my + N_DEV, N_DEV)
            rdma = pltpu.make_async_remote_copy(
                src_ref=out_ref.at[pl.ds(r0, CHB), :],
                dst_ref=stage_ref.at[t_dyn],
                send_sem=p1_send.at[t_dyn],
                recv_sem=p1_recv.at[t_dyn],
                device_id=(lax.rem(my + t_dyn, N_DEV),),
                device_id_type=pl.DeviceIdType.MESH,
            )

            @pl.when(j != my)
            def _(rdma=rdma):
                rdma.start()

            sends.append((j, rdma))

            @pl.when(j == my)
            def _(j=j, r0=r0):
                total = out_ref[r0:r0 + CHB, :]
                for t in range(1, N_DEV):
                    src = lax.rem(my - t + N_DEV, N_DEV)
                    recv = pltpu.make_async_remote_copy(
                        src_ref=out_ref.at[pl.ds(0, CHB), :],
                        dst_ref=stage_ref.at[t],
                        send_sem=p1_send.at[t],
                        recv_sem=p1_recv.at[t],
                        device_id=(src,),
                        device_id_type=pl.DeviceIdType.MESH,
                    )
                    recv.wait_recv()
                    total = total + stage_ref[t, :, :]
                out_ref[r0:r0 + CHB, :] = total
                for t in range(1, N_DEV):
                    b = pltpu.make_async_remote_copy(
                        src_ref=out_ref.at[pl.ds(r0, CHB), :],
                        dst_ref=out_ref.at[pl.ds(r0, CHB), :],
                        send_sem=p2_send.at[t],
                        recv_sem=p2_recv.at[t],
                        device_id=(lax.rem(my + t, N_DEV),),
                        device_id_type=pl.DeviceIdType.MESH,
                    )
                    b.start()

        for t in range(1, N_DEV):
            src = lax.rem(my - t + N_DEV, N_DEV)
            recv = pltpu.make_async_remote_copy(
                src_ref=out_ref.at[pl.ds(0, CHB), :],
                dst_ref=out_ref.at[pl.ds(src * CHB, CHB), :],
                send_sem=p2_send.at[t],
                recv_sem=p2_recv.at[t],
                device_id=(src,),
                device_id_type=pl.DeviceIdType.MESH,
            )
            recv.wait_recv()

        for j, rdma in sends:
            @pl.when(j != my)
            def _(rdma=rdma):
                rdma.wait_send()

        my_r0 = pl.ds(my * CHB, CHB)
        for t in range(1, N_DEV):
            d = pltpu.make_async_remote_copy(
                src_ref=out_ref.at[my_r0, :],
                dst_ref=out_ref.at[my_r0, :],
                send_sem=p2_send.at[t],
                recv_sem=p2_recv.at[t],
                device_id=(lax.rem(my + t, N_DEV),),
                device_id_type=pl.DeviceIdType.MESH,
            )
            d.wait_send()

    out = pl.pallas_call(
        body,
        out_shape=jax.ShapeDtypeStruct((SQ, D_MODEL), jnp.float32),
        in_specs=[pl.BlockSpec(memory_space=pltpu.VMEM)] * 5,
        out_specs=pl.BlockSpec(memory_space=pltpu.VMEM),
        scratch_shapes=[
            pltpu.VMEM((SQ, HQ_LOC * DH), jnp.float32),
            pltpu.VMEM((N_DEV, SQ // N_DEV, D_MODEL), jnp.float32),
            pltpu.SemaphoreType.DMA((N_DEV,)),
            pltpu.SemaphoreType.DMA((N_DEV,)),
            pltpu.SemaphoreType.DMA((N_DEV,)),
            pltpu.SemaphoreType.DMA((N_DEV,)),
        ],
        compiler_params=pltpu.CompilerParams(
            collective_id=0, vmem_limit_bytes=96 * 1024 * 1024),
    )(x2, Wq, k_loc, v_loc, Wo)
    return out[None]


# device time: 118755 ns/iter; 2.2916x vs baseline; 2.2916x over previous
import jax
import jax.numpy as jnp
from jax import lax
from jax.experimental import pallas as pl
from jax.experimental.pallas import tpu as pltpu

N_DEV = 8
HQ_LOC = 8
DH = 128
SQ = 1024
SKV = 1024
D_MODEL = 1024
SCALE = 0.08838834764831843


def kernel(x, Wq, K_ext, V_ext, Wo):
    pos = lax.axis_index("i")
    x2 = x[0]
    k_loc = lax.dynamic_slice_in_dim(K_ext[0], pos * HQ_LOC, HQ_LOC, axis=1)
    v_loc = lax.dynamic_slice_in_dim(V_ext[0], pos * HQ_LOC, HQ_LOC, axis=1)

    def body(x_ref, wq_ref, k_ref, v_ref, wo_ref, out_ref,
             ctx_ref, stage_ref, p1_send, p1_recv, p2_send, p2_recv):
        my = lax.axis_index("i")

        barrier_sem = pltpu.get_barrier_semaphore()
        for t in range(1, N_DEV):
            pl.semaphore_signal(
                barrier_sem, inc=1,
                device_id=(lax.rem(my + t, N_DEV),),
                device_id_type=pl.DeviceIdType.MESH,
            )
        pl.semaphore_wait(barrier_sem, N_DEV - 1)

        q = jnp.dot(x_ref[...], wq_ref[...], preferred_element_type=jnp.float32)

        QB = 128
        NEG = jnp.float32(-1e9)

        def band_bias(qb, k0, kw):
            qi = lax.broadcasted_iota(jnp.int32, (QB, kw), 0) + qb * QB
            ki = lax.broadcasted_iota(jnp.int32, (QB, kw), 1) + k0
            m = (jnp.abs(qi - ki) <= 128) | (ki < 32) | (qi < 32)
            return jnp.where(m, 0.0, NEG).astype(jnp.float32)

        def band_window(qb):
            lo = max(qb - 1, 0) * QB
            hi = min(qb + 2, N_DEV) * QB
            return lo, hi - lo

        biases = {}
        for qb in range(N_DEV):
            if qb == 0:
                biases[qb] = band_bias(0, 0, SKV)
            else:
                lo, w_ = band_window(qb)
                biases[qb] = (band_bias(qb, lo, w_),
                              None if qb == 1 else band_bias(qb, 0, 32))

        for h in range(HQ_LOC):
            q_h = q[:, h * DH:(h + 1) * DH]
            k_h = k_ref[:, h, :]
            v_h = v_ref[:, h, :]
            for qb in range(N_DEV):
                q_b = q_h[qb * QB:(qb + 1) * QB, :]
                if qb == 0:
                    s = lax.dot_general(
                        q_b, k_h, (((1,), (1,)), ((), ())),
                        preferred_element_type=jnp.float32,
                    ) * SCALE + biases[0]
                    m = jnp.max(s, axis=-1, keepdims=True)
                    w = jnp.exp(s - m)
                    den = jnp.sum(w, axis=-1, keepdims=True)
                    acc = jnp.dot(w, v_h, preferred_element_type=jnp.float32)
                else:
                    lo, w_ = band_window(qb)
                    bb, bg = biases[qb]
                    k_band = k_h[lo:lo + w_, :]
                    s_b = lax.dot_general(
                        q_b, k_band, (((1,), (1,)), ((), ())),
                        preferred_element_type=jnp.float32,
                    ) * SCALE + bb
                    m = jnp.max(s_b, axis=-1, keepdims=True)
                    if bg is not None:
                        s_g = lax.dot_general(
                            q_b, k_h[0:32, :], (((1,), (1,)), ((), ())),
                            preferred_element_type=jnp.float32,
                        ) * SCALE
                        m = jnp.maximum(m, jnp.max(s_g, axis=-1, keepdims=True))
                    w_b = jnp.exp(s_b - m)
                    den = jnp.sum(w_b, axis=-1, keepdims=True)
                    acc = jnp.dot(w_b, v_h[lo:lo + w_, :],
                                  preferred_element_type=jnp.float32)
                    if bg is not None:
                        w_g = jnp.exp(s_g - m)
                        den = den + jnp.sum(w_g, axis=-1, keepdims=True)
                        acc = acc + jnp.dot(w_g, v_h[0:32, :],
                                            preferred_element_type=jnp.float32)
                ctx_ref[qb * QB:(qb + 1) * QB, h * DH:(h + 1) * DH] = (
                    acc * (1.0 / den))

        CH = SQ // N_DEV
        sends = []
        for t in range(1, N_DEV):
            tgt = lax.rem(my + t, N_DEV)
            rows = pl.ds(tgt * CH, CH)
            out_ref[rows, :] = jnp.dot(
                ctx_ref[rows, :], wo_ref[...],
                preferred_element_type=jnp.float32)
            rdma = pltpu.make_async_remote_copy(
                src_ref=out_ref.at[rows, :],
                dst_ref=stage_ref.at[t],
                send_sem=p1_send.at[t],
                recv_sem=p1_recv.at[t],
                device_id=(tgt,),
                device_id_type=pl.DeviceIdType.MESH,
            )
            rdma.start()
            sends.append(rdma)

        my_rows = pl.ds(my * CH, CH)
        total = jnp.dot(ctx_ref[my_rows, :], wo_ref[...],
                        preferred_element_type=jnp.float32)
        for t in range(1, N_DEV):
            src = lax.rem(my - t + N_DEV, N_DEV)
            recv = pltpu.make_async_remote_copy(
                src_ref=out_ref.at[pl.ds(0, CH), :],
                dst_ref=stage_ref.at[t],
                send_sem=p1_send.at[t],
                recv_sem=p1_recv.at[t],
                device_id=(src,),
                device_id_type=pl.DeviceIdType.MESH,
            )
            recv.wait_recv()
            total = total + stage_ref[t, :, :]
        out_ref[my_rows, :] = total

        for t in range(1, N_DEV):
            tgt = lax.rem(my + t, N_DEV)
            rdma = pltpu.make_async_remote_copy(
                src_ref=out_ref.at[my_rows, :],
                dst_ref=out_ref.at[my_rows, :],
                send_sem=p2_send.at[t],
                recv_sem=p2_recv.at[t],
                device_id=(tgt,),
                device_id_type=pl.DeviceIdType.MESH,
            )
            rdma.start()
            sends.append(rdma)

        for t in range(1, N_DEV):
            src = lax.rem(my - t + N_DEV, N_DEV)
            recv = pltpu.make_async_remote_copy(
                src_ref=out_ref.at[pl.ds(0, CH), :],
                dst_ref=out_ref.at[pl.ds(src * CH, CH), :],
                send_sem=p2_send.at[t],
                recv_sem=p2_recv.at[t],
                device_id=(src,),
                device_id_type=pl.DeviceIdType.MESH,
            )
            recv.wait_recv()

        for rdma in sends:
            rdma.wait_send()

    out = pl.pallas_call(
        body,
        out_shape=jax.ShapeDtypeStruct((SQ, D_MODEL), jnp.float32),
        in_specs=[pl.BlockSpec(memory_space=pltpu.VMEM)] * 5,
        out_specs=pl.BlockSpec(memory_space=pltpu.VMEM),
        scratch_shapes=[
            pltpu.VMEM((SQ, HQ_LOC * DH), jnp.float32),
            pltpu.VMEM((N_DEV, SQ // N_DEV, D_MODEL), jnp.float32),
            pltpu.SemaphoreType.DMA((N_DEV,)),
            pltpu.SemaphoreType.DMA((N_DEV,)),
            pltpu.SemaphoreType.DMA((N_DEV,)),
            pltpu.SemaphoreType.DMA((N_DEV,)),
        ],
        compiler_params=pltpu.CompilerParams(
            collective_id=0, vmem_limit_bytes=96 * 1024 * 1024),
    )(x2, Wq, k_loc, v_loc, Wo)
    return out[None]


# device time: 81478 ns/iter; 3.3401x vs baseline; 1.4575x over previous
import jax
import jax.numpy as jnp
from jax import lax
from jax.experimental import pallas as pl
from jax.experimental.pallas import tpu as pltpu

N_DEV = 8
HQ_LOC = 8
DH = 128
SQ = 1024
SKV = 1024
D_MODEL = 1024
SCALE = 0.08838834764831843


def kernel(x, Wq, K_ext, V_ext, Wo):
    pos = lax.axis_index("i")
    x2 = x[0]
    k_loc = lax.dynamic_slice_in_dim(K_ext[0], pos * HQ_LOC, HQ_LOC, axis=1)
    v_loc = lax.dynamic_slice_in_dim(V_ext[0], pos * HQ_LOC, HQ_LOC, axis=1)

    def body(x_ref, wq_ref, k_ref, v_ref, wo_ref, out_ref,
             ctx_ref, pbuf_ref, stage_ref, bbuf_ref, rbuf_ref,
             p1_send, p1_recv, p2_send, p2_recv):
        my = lax.axis_index("i")

        barrier_sem = pltpu.get_barrier_semaphore()
        for t in range(1, N_DEV):
            pl.semaphore_signal(
                barrier_sem, inc=1,
                device_id=(lax.rem(my + t, N_DEV),),
                device_id_type=pl.DeviceIdType.MESH,
            )
        pl.semaphore_wait(barrier_sem, N_DEV - 1)

        q = jnp.dot(x_ref[...], wq_ref[...], preferred_element_type=jnp.float32)

        qi = lax.broadcasted_iota(jnp.int32, (SQ, SKV), 0)
        ki = lax.broadcasted_iota(jnp.int32, (SQ, SKV), 1)
        mask = (jnp.abs(qi - ki) <= 128) | (ki < 32) | (qi < 32)
        bias = jnp.where(mask, 0.0, -1e9).astype(jnp.float32)

        for h in range(HQ_LOC):
            q_h = q[:, h * DH:(h + 1) * DH]
            k_h = k_ref[:, h, :]
            v_h = v_ref[:, h, :]
            s = lax.dot_general(
                q_h, k_h, (((1,), (1,)), ((), ())),
                preferred_element_type=jnp.float32,
            ) * SCALE + bias
            m = jnp.max(s, axis=-1, keepdims=True)
            w = jnp.exp(s - m)
            w = w / jnp.sum(w, axis=-1, keepdims=True)
            ctx_ref[:, h * DH:(h + 1) * DH] = jnp.dot(
                w, v_h, preferred_element_type=jnp.float32)

        CH = SQ // N_DEV
        sends = []
        for t in range(1, N_DEV):
            tgt = lax.rem(my + t, N_DEV)
            rows = pl.ds(tgt * CH, CH)
            pbuf_ref[rows, :] = jnp.dot(
                ctx_ref[rows, :], wo_ref[...],
                preferred_element_type=jnp.float32).astype(jnp.bfloat16)
            rdma = pltpu.make_async_remote_copy(
                src_ref=pbuf_ref.at[rows, :],
                dst_ref=stage_ref.at[t],
                send_sem=p1_send.at[t],
                recv_sem=p1_recv.at[t],
                device_id=(tgt,),
                device_id_type=pl.DeviceIdType.MESH,
            )
            rdma.start()
            sends.append(rdma)

        my_rows = pl.ds(my * CH, CH)
        total = jnp.dot(ctx_ref[my_rows, :], wo_ref[...],
                        preferred_element_type=jnp.float32)
        for t in range(1, N_DEV):
            src = lax.rem(my - t + N_DEV, N_DEV)
            recv = pltpu.make_async_remote_copy(
                src_ref=pbuf_ref.at[pl.ds(0, CH), :],
                dst_ref=stage_ref.at[t],
                send_sem=p1_send.at[t],
                recv_sem=p1_recv.at[t],
                device_id=(src,),
                device_id_type=pl.DeviceIdType.MESH,
            )
            recv.wait_recv()
            total = total + stage_ref[t, :, :].astype(jnp.float32)
        out_ref[my_rows, :] = total
        bbuf_ref[...] = total.astype(jnp.bfloat16)

        for t in range(1, N_DEV):
            tgt = lax.rem(my + t, N_DEV)
            rdma = pltpu.make_async_remote_copy(
                src_ref=bbuf_ref,
                dst_ref=rbuf_ref.at[t],
                send_sem=p2_send.at[t],
                recv_sem=p2_recv.at[t],
                device_id=(tgt,),
                device_id_type=pl.DeviceIdType.MESH,
            )
            rdma.start()
            sends.append(rdma)

        for t in range(1, N_DEV):
            src = lax.rem(my - t + N_DEV, N_DEV)
            recv = pltpu.make_async_remote_copy(
                src_ref=bbuf_ref,
                dst_ref=rbuf_ref.at[t],
                send_sem=p2_send.at[t],
                recv_sem=p2_recv.at[t],
                device_id=(src,),
                device_id_type=pl.DeviceIdType.MESH,
            )
            recv.wait_recv()
            out_ref[pl.ds(src * CH, CH), :] = (
                rbuf_ref[t, :, :].astype(jnp.float32))

        for rdma in sends:
            rdma.wait_send()

    out = pl.pallas_call(
        body,
        out_shape=jax.ShapeDtypeStruct((SQ, D_MODEL), jnp.float32),
        in_specs=[pl.BlockSpec(memory_space=pltpu.VMEM)] * 5,
        out_specs=pl.BlockSpec(memory_space=pltpu.VMEM),
        scratch_shapes=[
            pltpu.VMEM((SQ, HQ_LOC * DH), jnp.float32),
            pltpu.VMEM((SQ, D_MODEL), jnp.bfloat16),
            pltpu.VMEM((N_DEV, SQ // N_DEV, D_MODEL), jnp.bfloat16),
            pltpu.VMEM((SQ // N_DEV, D_MODEL), jnp.bfloat16),
            pltpu.VMEM((N_DEV, SQ // N_DEV, D_MODEL), jnp.bfloat16),
            pltpu.SemaphoreType.DMA((N_DEV,)),
            pltpu.SemaphoreType.DMA((N_DEV,)),
            pltpu.SemaphoreType.DMA((N_DEV,)),
            pltpu.SemaphoreType.DMA((N_DEV,)),
        ],
        compiler_params=pltpu.CompilerParams(
            collective_id=0, vmem_limit_bytes=96 * 1024 * 1024),
    )(x2, Wq, k_loc, v_loc, Wo)
    return out[None]
